# Initial kernel scaffold; baseline (speedup 1.0000x reference)
import functools

import jax
import jax.numpy as jnp
from jax import lax
from jax.experimental import pallas as pl
from jax.experimental.pallas import tpu as pltpu
from jax.experimental.pallas import tpu_sc as plsc

N_NODES = 10000
N_EDGES = 320000
IN_FEATS = 128
HIDDEN = 128
NUM_CLASSES = 40
CLS_PAD = 48

NC = 2
NS = 16
NW = NC * NS
EDGES_PER_TILE = N_EDGES // NW
CHUNK = 80
NCHUNKS = EDGES_PER_TILE // CHUNK
ROWS_PER_TILE = N_NODES // NS

_mesh = plsc.VectorSubcoreMesh(core_axis_name="c", subcore_axis_name="s")
_f32 = jnp.float32


@functools.partial(
    pl.kernel,
    out_type=[
        jax.ShapeDtypeStruct((NC, N_NODES, 16), jnp.float32),
        jax.ShapeDtypeStruct((NC, N_NODES, 16), jnp.float32),
    ],
    mesh=_mesh,
    scratch_types=[
        pltpu.VMEM((CHUNK,), jnp.int32),
        pltpu.VMEM((CHUNK,), jnp.int32),
        pltpu.VMEM((CHUNK, 16), jnp.float32),
        pltpu.VMEM_SHARED((N_NODES, 16), jnp.float32),
        pltpu.VMEM_SHARED((N_NODES, 16), jnp.float32),
    ],
)
def _sc_degrees(src_hbm, dst_hbm, ones_hbm, zeros_hbm, od_out, id_out,
                sidx, didx, ones_v, od_sh, id_sh):
    c = lax.axis_index("c")
    s = lax.axis_index("s")
    wid = s * NC + c

    pltpu.sync_copy(ones_hbm, ones_v)
    row0 = s * ROWS_PER_TILE
    pltpu.sync_copy(zeros_hbm, od_sh.at[pl.ds(row0, ROWS_PER_TILE)])
    pltpu.sync_copy(zeros_hbm, id_sh.at[pl.ds(row0, ROWS_PER_TILE)])
    plsc.subcore_barrier()

    @pl.loop(0, NCHUNKS)
    def _(j):
        base = wid * EDGES_PER_TILE + j * CHUNK
        pltpu.sync_copy(src_hbm.at[pl.ds(base, CHUNK)], sidx)
        pltpu.sync_copy(dst_hbm.at[pl.ds(base, CHUNK)], didx)
        pltpu.sync_copy(ones_v, od_sh.at[sidx], add=True)
        pltpu.sync_copy(ones_v, id_sh.at[didx], add=True)

    plsc.subcore_barrier()
    pltpu.sync_copy(od_sh.at[pl.ds(row0, ROWS_PER_TILE)],
                    od_out.at[c, pl.ds(row0, ROWS_PER_TILE)])
    pltpu.sync_copy(id_sh.at[pl.ds(row0, ROWS_PER_TILE)],
                    id_out.at[c, pl.ds(row0, ROWS_PER_TILE)])


def _make_sc_aggregate(width):
    @functools.partial(
        pl.kernel,
        out_type=jax.ShapeDtypeStruct((NC, N_NODES, width), jnp.float32),
        mesh=_mesh,
        scratch_types=[
            pltpu.VMEM((CHUNK,), jnp.int32),
            pltpu.VMEM((CHUNK,), jnp.int32),
            pltpu.VMEM((CHUNK, width), jnp.float32),
            pltpu.VMEM_SHARED((N_NODES, width), jnp.float32),
            pltpu.SemaphoreType.DMA,
        ],
    )
    def _sc_aggregate(h_hbm, src_hbm, dst_hbm, zeros_hbm, out_hbm,
                      sidx, didx, rows, agg_sh, sem):
        c = lax.axis_index("c")
        s = lax.axis_index("s")
        wid = s * NC + c

        row0 = s * ROWS_PER_TILE
        pltpu.sync_copy(zeros_hbm, agg_sh.at[pl.ds(row0, ROWS_PER_TILE)])
        plsc.subcore_barrier()

        @pl.loop(0, NCHUNKS)
        def _(j):
            base = wid * EDGES_PER_TILE + j * CHUNK
            pltpu.sync_copy(src_hbm.at[pl.ds(base, CHUNK)], sidx)
            pltpu.sync_copy(dst_hbm.at[pl.ds(base, CHUNK)], didx)
            pltpu.async_copy(h_hbm.at[sidx], rows, sem).wait()
            pltpu.sync_copy(rows, agg_sh.at[didx], add=True)

        plsc.subcore_barrier()
        pltpu.sync_copy(agg_sh.at[pl.ds(row0, ROWS_PER_TILE)],
                        out_hbm.at[c, pl.ds(row0, ROWS_PER_TILE)])

    return _sc_aggregate


_sc_aggregate_h = _make_sc_aggregate(HIDDEN)
_sc_aggregate_c = _make_sc_aggregate(CLS_PAD)


_ROWS_BLK = 1000
_GRID = N_NODES // _ROWS_BLK


def _norm_from_partials(p_ref):
    deg = p_ref[0][:, :1] + p_ref[1][:, :1]
    return lax.rsqrt(jnp.maximum(deg, 1.0))


def _tc_scale_body(feat_ref, odp_ref, h1_ref):
    h1_ref[...] = feat_ref[...] * _norm_from_partials(odp_ref)


def _tc_scale(features, odeg_p):
    return pl.pallas_call(
        _tc_scale_body,
        grid=(_GRID,),
        in_specs=[
            pl.BlockSpec((_ROWS_BLK, IN_FEATS), lambda i: (i, 0)),
            pl.BlockSpec((NC, _ROWS_BLK, 16), lambda i: (0, i, 0)),
        ],
        out_specs=pl.BlockSpec((_ROWS_BLK, IN_FEATS), lambda i: (i, 0)),
        out_shape=jax.ShapeDtypeStruct((N_NODES, IN_FEATS), jnp.float32),
    )(features, odeg_p)


def _tc_layer1_body(p1_ref, idp_ref, odp_ref, w1_ref, b1_ref, w2_ref, y_ref):
    agg = (p1_ref[0] + p1_ref[1]) * _norm_from_partials(idp_ref)
    x1 = jnp.dot(agg, w1_ref[...], preferred_element_type=jnp.float32,
                 precision=lax.Precision.HIGHEST)
    x1 = jnp.maximum(x1 + b1_ref[...], 0.0)
    x1 = x1 * _norm_from_partials(odp_ref)
    y_ref[...] = jnp.dot(x1, w2_ref[...], preferred_element_type=jnp.float32,
                         precision=lax.Precision.HIGHEST)


def _tc_layer1(p1, ideg_p, odeg_p, W1, b1, W2p):
    return pl.pallas_call(
        _tc_layer1_body,
        grid=(_GRID,),
        in_specs=[
            pl.BlockSpec((NC, _ROWS_BLK, HIDDEN), lambda i: (0, i, 0)),
            pl.BlockSpec((NC, _ROWS_BLK, 16), lambda i: (0, i, 0)),
            pl.BlockSpec((NC, _ROWS_BLK, 16), lambda i: (0, i, 0)),
            pl.BlockSpec((IN_FEATS, HIDDEN), lambda i: (0, 0)),
            pl.BlockSpec((1, HIDDEN), lambda i: (0, 0)),
            pl.BlockSpec((HIDDEN, CLS_PAD), lambda i: (0, 0)),
        ],
        out_specs=pl.BlockSpec((_ROWS_BLK, CLS_PAD), lambda i: (i, 0)),
        out_shape=jax.ShapeDtypeStruct((N_NODES, CLS_PAD), jnp.float32),
    )(p1, ideg_p, odeg_p, W1, b1, W2p)


def _tc_layer2_body(p2_ref, idp_ref, b2_ref, out_ref):
    agg = (p2_ref[0] + p2_ref[1])[:, :NUM_CLASSES]
    out_ref[...] = agg * _norm_from_partials(idp_ref) + b2_ref[...]


def _tc_layer2(p2, ideg_p, b2):
    return pl.pallas_call(
        _tc_layer2_body,
        grid=(_GRID,),
        in_specs=[
            pl.BlockSpec((NC, _ROWS_BLK, CLS_PAD), lambda i: (0, i, 0)),
            pl.BlockSpec((NC, _ROWS_BLK, 16), lambda i: (0, i, 0)),
            pl.BlockSpec((1, NUM_CLASSES), lambda i: (0, 0)),
        ],
        out_specs=pl.BlockSpec((_ROWS_BLK, NUM_CLASSES), lambda i: (i, 0)),
        out_shape=jax.ShapeDtypeStruct((N_NODES, NUM_CLASSES), jnp.float32),
    )(p2, ideg_p, b2)


def kernel(features, edge_index, W1, b1, W2, b2):
    src = edge_index[0]
    dst = edge_index[1]

    ones16 = jnp.ones((CHUNK, 16), jnp.float32)
    zeros16 = jnp.zeros((ROWS_PER_TILE, 16), jnp.float32)
    zeros_h = jnp.zeros((ROWS_PER_TILE, HIDDEN), jnp.float32)
    zeros_c = jnp.zeros((ROWS_PER_TILE, CLS_PAD), jnp.float32)
    W2p = jnp.pad(W2, ((0, 0), (0, CLS_PAD - NUM_CLASSES)))

    odeg_p, ideg_p = _sc_degrees(src, dst, ones16, zeros16)

    h1 = _tc_scale(features, odeg_p)
    p1 = _sc_aggregate_h(h1, src, dst, zeros_h)
    y = _tc_layer1(p1, ideg_p, odeg_p, W1, b1.reshape(1, HIDDEN), W2p)
    p2 = _sc_aggregate_c(y, src, dst, zeros_c)
    out = _tc_layer2(p2, ideg_p, b2.reshape(1, NUM_CLASSES))
    return out


# speedup vs baseline: 4.5091x; 4.5091x over previous
import functools

import jax
import jax.numpy as jnp
from jax import lax
from jax.experimental import pallas as pl
from jax.experimental.pallas import tpu as pltpu
from jax.experimental.pallas import tpu_sc as plsc

N_NODES = 10000
N_EDGES = 320000
IN_FEATS = 128
HIDDEN = 128
NUM_CLASSES = 40
CLS_PAD = 48

NC = 2
NS = 16
NW = NC * NS
EDGES_PER_TILE = N_EDGES // NW
CHUNK = 80
NCHUNKS = EDGES_PER_TILE // CHUNK
N_PAD = 10240
ROWS_PER_TILE = N_PAD // NS

_mesh = plsc.VectorSubcoreMesh(core_axis_name="c", subcore_axis_name="s")
_f32 = jnp.float32


@functools.partial(
    pl.kernel,
    out_type=[
        jax.ShapeDtypeStruct((NC, N_PAD, 16), jnp.float32),
        jax.ShapeDtypeStruct((NC, N_PAD, 16), jnp.float32),
    ],
    mesh=_mesh,
    scratch_types=[
        pltpu.VMEM((CHUNK,), jnp.int32),
        pltpu.VMEM((CHUNK,), jnp.int32),
        pltpu.VMEM((CHUNK, 16), jnp.float32),
        pltpu.VMEM_SHARED((N_PAD, 16), jnp.float32),
        pltpu.VMEM_SHARED((N_PAD, 16), jnp.float32),
    ],
    compiler_params=pltpu.CompilerParams(use_tc_tiling_on_sc=False),
)
def _sc_degrees(src_hbm, dst_hbm, ones_hbm, zeros_hbm, od_out, id_out,
                sidx, didx, ones_v, od_sh, id_sh):
    c = lax.axis_index("c")
    s = lax.axis_index("s")
    wid = s * NC + c

    pltpu.sync_copy(ones_hbm, ones_v)
    row0 = s * ROWS_PER_TILE
    pltpu.sync_copy(zeros_hbm, od_sh.at[pl.ds(row0, ROWS_PER_TILE)])
    pltpu.sync_copy(zeros_hbm, id_sh.at[pl.ds(row0, ROWS_PER_TILE)])
    plsc.subcore_barrier()

    @pl.loop(0, NCHUNKS)
    def _(j):
        base = wid * EDGES_PER_TILE + j * CHUNK
        pltpu.sync_copy(src_hbm.at[pl.ds(base, CHUNK)], sidx)
        pltpu.sync_copy(dst_hbm.at[pl.ds(base, CHUNK)], didx)
        pltpu.sync_copy(ones_v, od_sh.at[sidx], add=True)
        pltpu.sync_copy(ones_v, id_sh.at[didx], add=True)

    plsc.subcore_barrier()
    pltpu.sync_copy(od_sh.at[pl.ds(row0, ROWS_PER_TILE)],
                    od_out.at[c, pl.ds(row0, ROWS_PER_TILE)])
    pltpu.sync_copy(id_sh.at[pl.ds(row0, ROWS_PER_TILE)],
                    id_out.at[c, pl.ds(row0, ROWS_PER_TILE)])


def _make_sc_aggregate(width):
    @functools.partial(
        pl.kernel,
        out_type=jax.ShapeDtypeStruct((NC, N_PAD, width), jnp.float32),
        mesh=_mesh,
        scratch_types=[
            pltpu.VMEM((CHUNK,), jnp.int32),
            pltpu.VMEM((CHUNK,), jnp.int32),
            pltpu.VMEM((CHUNK, width), jnp.float32),
            pltpu.VMEM_SHARED((N_PAD, width), jnp.float32),
            pltpu.SemaphoreType.DMA,
        ],
        compiler_params=pltpu.CompilerParams(use_tc_tiling_on_sc=False),
    )
    def _sc_aggregate(h_hbm, src_hbm, dst_hbm, zeros_hbm, out_hbm,
                      sidx, didx, rows, agg_sh, sem):
        c = lax.axis_index("c")
        s = lax.axis_index("s")
        wid = s * NC + c

        row0 = s * ROWS_PER_TILE
        pltpu.sync_copy(zeros_hbm, agg_sh.at[pl.ds(row0, ROWS_PER_TILE)])
        plsc.subcore_barrier()

        @pl.loop(0, NCHUNKS)
        def _(j):
            base = wid * EDGES_PER_TILE + j * CHUNK
            pltpu.sync_copy(src_hbm.at[pl.ds(base, CHUNK)], sidx)
            pltpu.sync_copy(dst_hbm.at[pl.ds(base, CHUNK)], didx)
            pltpu.async_copy(h_hbm.at[sidx], rows, sem).wait()
            pltpu.sync_copy(rows, agg_sh.at[didx], add=True)

        plsc.subcore_barrier()
        pltpu.sync_copy(agg_sh.at[pl.ds(row0, ROWS_PER_TILE)],
                        out_hbm.at[c, pl.ds(row0, ROWS_PER_TILE)])

    return _sc_aggregate


_sc_aggregate_h = _make_sc_aggregate(HIDDEN)
_sc_aggregate_c = _make_sc_aggregate(CLS_PAD)


_ROWS_BLK = 1000
_GRID = N_NODES // _ROWS_BLK


def _norm_from_partials(p_ref):
    deg = p_ref[0][:, :1] + p_ref[1][:, :1]
    return lax.rsqrt(jnp.maximum(deg, 1.0))


def _tc_scale_body(feat_ref, odp_ref, h1_ref):
    h1_ref[...] = feat_ref[...] * _norm_from_partials(odp_ref)


def _tc_scale(features, odeg_p):
    return pl.pallas_call(
        _tc_scale_body,
        grid=(_GRID,),
        in_specs=[
            pl.BlockSpec((_ROWS_BLK, IN_FEATS), lambda i: (i, 0)),
            pl.BlockSpec((NC, _ROWS_BLK, 16), lambda i: (0, i, 0)),
        ],
        out_specs=pl.BlockSpec((_ROWS_BLK, IN_FEATS), lambda i: (i, 0)),
        out_shape=jax.ShapeDtypeStruct((N_NODES, IN_FEATS), jnp.float32),
    )(features, odeg_p)


def _tc_layer1_body(p1_ref, idp_ref, odp_ref, w1_ref, b1_ref, w2_ref, y_ref):
    agg = (p1_ref[0] + p1_ref[1]) * _norm_from_partials(idp_ref)
    x1 = jnp.dot(agg, w1_ref[...], preferred_element_type=jnp.float32,
                 precision=lax.Precision.HIGHEST)
    x1 = jnp.maximum(x1 + b1_ref[...], 0.0)
    x1 = x1 * _norm_from_partials(odp_ref)
    y_ref[...] = jnp.dot(x1, w2_ref[...], preferred_element_type=jnp.float32,
                         precision=lax.Precision.HIGHEST)


def _tc_layer1(p1, ideg_p, odeg_p, W1, b1, W2p):
    return pl.pallas_call(
        _tc_layer1_body,
        grid=(_GRID,),
        in_specs=[
            pl.BlockSpec((NC, _ROWS_BLK, HIDDEN), lambda i: (0, i, 0)),
            pl.BlockSpec((NC, _ROWS_BLK, 16), lambda i: (0, i, 0)),
            pl.BlockSpec((NC, _ROWS_BLK, 16), lambda i: (0, i, 0)),
            pl.BlockSpec((IN_FEATS, HIDDEN), lambda i: (0, 0)),
            pl.BlockSpec((1, HIDDEN), lambda i: (0, 0)),
            pl.BlockSpec((HIDDEN, CLS_PAD), lambda i: (0, 0)),
        ],
        out_specs=pl.BlockSpec((_ROWS_BLK, CLS_PAD), lambda i: (i, 0)),
        out_shape=jax.ShapeDtypeStruct((N_NODES, CLS_PAD), jnp.float32),
    )(p1, ideg_p, odeg_p, W1, b1, W2p)


def _tc_layer2_body(p2_ref, idp_ref, b2_ref, out_ref):
    agg = (p2_ref[0] + p2_ref[1])[:, :NUM_CLASSES]
    out_ref[...] = agg * _norm_from_partials(idp_ref) + b2_ref[...]


def _tc_layer2(p2, ideg_p, b2):
    return pl.pallas_call(
        _tc_layer2_body,
        grid=(_GRID,),
        in_specs=[
            pl.BlockSpec((NC, _ROWS_BLK, CLS_PAD), lambda i: (0, i, 0)),
            pl.BlockSpec((NC, _ROWS_BLK, 16), lambda i: (0, i, 0)),
            pl.BlockSpec((1, NUM_CLASSES), lambda i: (0, 0)),
        ],
        out_specs=pl.BlockSpec((_ROWS_BLK, NUM_CLASSES), lambda i: (i, 0)),
        out_shape=jax.ShapeDtypeStruct((N_NODES, NUM_CLASSES), jnp.float32),
    )(p2, ideg_p, b2)


def kernel(features, edge_index, W1, b1, W2, b2):
    src = edge_index[0]
    dst = edge_index[1]

    ones16 = jnp.ones((CHUNK, 16), jnp.float32)
    zeros16 = jnp.zeros((ROWS_PER_TILE, 16), jnp.float32)
    zeros_h = jnp.zeros((ROWS_PER_TILE, HIDDEN), jnp.float32)
    zeros_c = jnp.zeros((ROWS_PER_TILE, CLS_PAD), jnp.float32)
    W2p = jnp.pad(W2, ((0, 0), (0, CLS_PAD - NUM_CLASSES)))

    odeg_p, ideg_p = _sc_degrees(src, dst, ones16, zeros16)

    h1 = _tc_scale(features, odeg_p)
    p1 = _sc_aggregate_h(h1, src, dst, zeros_h)
    y = _tc_layer1(p1, ideg_p, odeg_p, W1, b1.reshape(1, HIDDEN), W2p)
    p2 = _sc_aggregate_c(y, src, dst, zeros_c)
    out = _tc_layer2(p2, ideg_p, b2.reshape(1, NUM_CLASSES))
    return out


# speedup vs baseline: 8.8398x; 1.9604x over previous
import functools

import jax
import jax.numpy as jnp
from jax import lax
from jax.experimental import pallas as pl
from jax.experimental.pallas import tpu as pltpu
from jax.experimental.pallas import tpu_sc as plsc

N_NODES = 10000
N_EDGES = 320000
IN_FEATS = 128
HIDDEN = 128
NUM_CLASSES = 40
CLS_PAD = 48

NC = 2
NS = 16
NW = NC * NS
EDGES_PER_TILE = N_EDGES // NW
CHUNK = 80
NCHUNKS = EDGES_PER_TILE // CHUNK
N_PAD = 10240
ROWS_PER_TILE = N_PAD // NS

_mesh = plsc.VectorSubcoreMesh(core_axis_name="c", subcore_axis_name="s")
_sc_params = pltpu.CompilerParams(use_tc_tiling_on_sc=False)


@functools.partial(
    pl.kernel,
    out_type=[
        jax.ShapeDtypeStruct((NC, N_PAD, 16), jnp.float32),
        jax.ShapeDtypeStruct((NC, N_PAD, 16), jnp.float32),
    ],
    mesh=_mesh,
    scratch_types=[
        pltpu.VMEM((NCHUNKS, CHUNK), jnp.int32),
        pltpu.VMEM((NCHUNKS, CHUNK), jnp.int32),
        pltpu.VMEM((CHUNK, 16), jnp.float32),
        pltpu.VMEM_SHARED((N_PAD, 16), jnp.float32),
        pltpu.VMEM_SHARED((N_PAD, 16), jnp.float32),
        pltpu.SemaphoreType.DMA,
        pltpu.SemaphoreType.DMA,
    ],
    compiler_params=_sc_params,
)
def _sc_degrees(src_hbm, dst_hbm, ones_hbm, zeros_hbm, od_out, id_out,
                sidx, didx, ones_v, od_sh, id_sh, sem_o, sem_i):
    c = lax.axis_index("c")
    s = lax.axis_index("s")
    wid = s * NC + c

    pltpu.sync_copy(src_hbm.at[wid], sidx)
    pltpu.sync_copy(dst_hbm.at[wid], didx)
    pltpu.sync_copy(ones_hbm, ones_v)
    row0 = s * ROWS_PER_TILE
    pltpu.sync_copy(zeros_hbm, od_sh.at[pl.ds(row0, ROWS_PER_TILE)])
    pltpu.sync_copy(zeros_hbm, id_sh.at[pl.ds(row0, ROWS_PER_TILE)])
    plsc.subcore_barrier()

    @pl.loop(0, NCHUNKS)
    def _(j):
        pltpu.async_copy(ones_v, od_sh.at[sidx.at[j]], sem_o, add=True)
        pltpu.async_copy(ones_v, id_sh.at[didx.at[j]], sem_i, add=True)

        @pl.when(j >= 4)
        def _():
            pltpu.make_async_copy(ones_v, od_sh.at[sidx.at[j - 4]], sem_o).wait()
            pltpu.make_async_copy(ones_v, id_sh.at[didx.at[j - 4]], sem_i).wait()

    @pl.loop(NCHUNKS - 4, NCHUNKS)
    def _(j):
        pltpu.make_async_copy(ones_v, od_sh.at[sidx.at[j]], sem_o).wait()
        pltpu.make_async_copy(ones_v, id_sh.at[didx.at[j]], sem_i).wait()

    plsc.subcore_barrier()
    pltpu.sync_copy(od_sh.at[pl.ds(row0, ROWS_PER_TILE)],
                    od_out.at[c, pl.ds(row0, ROWS_PER_TILE)])
    pltpu.sync_copy(id_sh.at[pl.ds(row0, ROWS_PER_TILE)],
                    id_out.at[c, pl.ds(row0, ROWS_PER_TILE)])


def _make_sc_aggregate(width):
    @functools.partial(
        pl.kernel,
        out_type=jax.ShapeDtypeStruct((NC, N_PAD, width), jnp.float32),
        mesh=_mesh,
        scratch_types=[
            pltpu.VMEM((NCHUNKS, CHUNK), jnp.int32),
            pltpu.VMEM((NCHUNKS, CHUNK), jnp.int32),
            pltpu.VMEM((CHUNK, width), jnp.float32),
            pltpu.VMEM((CHUNK, width), jnp.float32),
            pltpu.VMEM_SHARED((N_PAD, width), jnp.float32),
            pltpu.SemaphoreType.DMA,
            pltpu.SemaphoreType.DMA,
        ],
        compiler_params=_sc_params,
    )
    def _sc_aggregate(h_hbm, src_hbm, dst_hbm, zeros_hbm, out_hbm,
                      sidx, didx, buf0, buf1, agg_sh, gsem0, gsem1):
        c = lax.axis_index("c")
        s = lax.axis_index("s")
        wid = s * NC + c

        pltpu.sync_copy(src_hbm.at[wid], sidx)
        pltpu.sync_copy(dst_hbm.at[wid], didx)
        row0 = s * ROWS_PER_TILE
        pltpu.sync_copy(zeros_hbm, agg_sh.at[pl.ds(row0, ROWS_PER_TILE)])
        plsc.subcore_barrier()

        pltpu.async_copy(h_hbm.at[sidx.at[0]], buf0, gsem0)

        @pl.loop(0, NCHUNKS - 1, step=2)
        def _(j):
            pltpu.make_async_copy(h_hbm.at[sidx.at[j]], buf0, gsem0).wait()
            pltpu.async_copy(h_hbm.at[sidx.at[j + 1]], buf1, gsem1)
            pltpu.sync_copy(buf0, agg_sh.at[didx.at[j]], add=True)
            pltpu.make_async_copy(h_hbm.at[sidx.at[j + 1]], buf1, gsem1).wait()

            @pl.when(j + 2 < NCHUNKS)
            def _():
                pltpu.async_copy(h_hbm.at[sidx.at[j + 2]], buf0, gsem0)

            pltpu.sync_copy(buf1, agg_sh.at[didx.at[j + 1]], add=True)

        pltpu.make_async_copy(h_hbm.at[sidx.at[NCHUNKS - 1]], buf0, gsem0).wait()
        pltpu.sync_copy(buf0, agg_sh.at[didx.at[NCHUNKS - 1]], add=True)

        plsc.subcore_barrier()
        pltpu.sync_copy(agg_sh.at[pl.ds(row0, ROWS_PER_TILE)],
                        out_hbm.at[c, pl.ds(row0, ROWS_PER_TILE)])

    return _sc_aggregate


_sc_aggregate_h = _make_sc_aggregate(HIDDEN)
_sc_aggregate_c = _make_sc_aggregate(CLS_PAD)


_ROWS_BLK = 1000
_GRID = N_NODES // _ROWS_BLK


def _norm_from_partials(p_ref):
    deg = p_ref[0][:, :1] + p_ref[1][:, :1]
    return lax.rsqrt(jnp.maximum(deg, 1.0))


def _tc_scale_body(feat_ref, odp_ref, h1_ref):
    h1_ref[...] = feat_ref[...] * _norm_from_partials(odp_ref)


def _tc_scale(features, odeg_p):
    return pl.pallas_call(
        _tc_scale_body,
        grid=(_GRID,),
        in_specs=[
            pl.BlockSpec((_ROWS_BLK, IN_FEATS), lambda i: (i, 0)),
            pl.BlockSpec((NC, _ROWS_BLK, 16), lambda i: (0, i, 0)),
        ],
        out_specs=pl.BlockSpec((_ROWS_BLK, IN_FEATS), lambda i: (i, 0)),
        out_shape=jax.ShapeDtypeStruct((N_NODES, IN_FEATS), jnp.float32),
    )(features, odeg_p)


def _tc_layer1_body(p1_ref, idp_ref, odp_ref, w1_ref, b1_ref, w2_ref, y_ref):
    agg = (p1_ref[0] + p1_ref[1]) * _norm_from_partials(idp_ref)
    x1 = jnp.dot(agg, w1_ref[...], preferred_element_type=jnp.float32,
                 precision=lax.Precision.HIGHEST)
    x1 = jnp.maximum(x1 + b1_ref[...], 0.0)
    x1 = x1 * _norm_from_partials(odp_ref)
    y_ref[...] = jnp.dot(x1, w2_ref[...], preferred_element_type=jnp.float32,
                         precision=lax.Precision.HIGHEST)


def _tc_layer1(p1, ideg_p, odeg_p, W1, b1, W2p):
    return pl.pallas_call(
        _tc_layer1_body,
        grid=(_GRID,),
        in_specs=[
            pl.BlockSpec((NC, _ROWS_BLK, HIDDEN), lambda i: (0, i, 0)),
            pl.BlockSpec((NC, _ROWS_BLK, 16), lambda i: (0, i, 0)),
            pl.BlockSpec((NC, _ROWS_BLK, 16), lambda i: (0, i, 0)),
            pl.BlockSpec((IN_FEATS, HIDDEN), lambda i: (0, 0)),
            pl.BlockSpec((1, HIDDEN), lambda i: (0, 0)),
            pl.BlockSpec((HIDDEN, CLS_PAD), lambda i: (0, 0)),
        ],
        out_specs=pl.BlockSpec((_ROWS_BLK, CLS_PAD), lambda i: (i, 0)),
        out_shape=jax.ShapeDtypeStruct((N_NODES, CLS_PAD), jnp.float32),
    )(p1, ideg_p, odeg_p, W1, b1, W2p)


def _tc_layer2_body(p2_ref, idp_ref, b2_ref, out_ref):
    agg = (p2_ref[0] + p2_ref[1])[:, :NUM_CLASSES]
    out_ref[...] = agg * _norm_from_partials(idp_ref) + b2_ref[...]


def _tc_layer2(p2, ideg_p, b2):
    return pl.pallas_call(
        _tc_layer2_body,
        grid=(_GRID,),
        in_specs=[
            pl.BlockSpec((NC, _ROWS_BLK, CLS_PAD), lambda i: (0, i, 0)),
            pl.BlockSpec((NC, _ROWS_BLK, 16), lambda i: (0, i, 0)),
            pl.BlockSpec((1, NUM_CLASSES), lambda i: (0, 0)),
        ],
        out_specs=pl.BlockSpec((_ROWS_BLK, NUM_CLASSES), lambda i: (i, 0)),
        out_shape=jax.ShapeDtypeStruct((N_NODES, NUM_CLASSES), jnp.float32),
    )(p2, ideg_p, b2)


def kernel(features, edge_index, W1, b1, W2, b2):
    src = edge_index[0].reshape(NW, NCHUNKS, CHUNK)
    dst = edge_index[1].reshape(NW, NCHUNKS, CHUNK)

    ones16 = jnp.ones((CHUNK, 16), jnp.float32)
    zeros16 = jnp.zeros((ROWS_PER_TILE, 16), jnp.float32)
    zeros_h = jnp.zeros((ROWS_PER_TILE, HIDDEN), jnp.float32)
    zeros_c = jnp.zeros((ROWS_PER_TILE, CLS_PAD), jnp.float32)
    W2p = jnp.pad(W2, ((0, 0), (0, CLS_PAD - NUM_CLASSES)))

    odeg_p, ideg_p = _sc_degrees(src, dst, ones16, zeros16)

    h1 = _tc_scale(features, odeg_p)
    p1 = _sc_aggregate_h(h1, src, dst, zeros_h)
    y = _tc_layer1(p1, ideg_p, odeg_p, W1, b1.reshape(1, HIDDEN), W2p)
    p2 = _sc_aggregate_c(y, src, dst, zeros_c)
    out = _tc_layer2(p2, ideg_p, b2.reshape(1, NUM_CLASSES))
    return out


# speedup vs baseline: 9.5181x; 1.0767x over previous
import functools

import jax
import jax.numpy as jnp
from jax import lax
from jax.experimental import pallas as pl
from jax.experimental.pallas import tpu as pltpu
from jax.experimental.pallas import tpu_sc as plsc

N_NODES = 10000
N_EDGES = 320000
IN_FEATS = 128
HIDDEN = 128
NUM_CLASSES = 40
CLS_PAD = 48

NC = 2
NS = 16
NW = NC * NS
EDGES_PER_TILE = N_EDGES // NW
CHUNK = 80
NCHUNKS = EDGES_PER_TILE // CHUNK
N_PAD = 10240
ROWS_PER_TILE = N_PAD // NS

_mesh = plsc.VectorSubcoreMesh(core_axis_name="c", subcore_axis_name="s")
_sc_params = pltpu.CompilerParams(use_tc_tiling_on_sc=False)


@functools.partial(
    pl.kernel,
    out_type=[
        jax.ShapeDtypeStruct((NC, N_PAD, 16), jnp.float32),
        jax.ShapeDtypeStruct((NC, N_PAD, 16), jnp.float32),
    ],
    mesh=_mesh,
    scratch_types=[
        pltpu.VMEM((NCHUNKS, CHUNK), jnp.int32),
        pltpu.VMEM((NCHUNKS, CHUNK), jnp.int32),
        pltpu.VMEM((CHUNK, 16), jnp.float32),
        pltpu.VMEM_SHARED((N_PAD, 16), jnp.float32),
        pltpu.VMEM_SHARED((N_PAD, 16), jnp.float32),
        pltpu.SemaphoreType.DMA,
        pltpu.SemaphoreType.DMA,
    ],
    compiler_params=_sc_params,
)
def _sc_degrees(src_hbm, dst_hbm, ones_hbm, zeros_hbm, od_out, id_out,
                sidx, didx, ones_v, od_sh, id_sh, sem_o, sem_i):
    c = lax.axis_index("c")
    s = lax.axis_index("s")
    wid = s * NC + c

    pltpu.sync_copy(src_hbm.at[wid], sidx)
    pltpu.sync_copy(dst_hbm.at[wid], didx)
    pltpu.sync_copy(ones_hbm, ones_v)
    row0 = s * ROWS_PER_TILE
    pltpu.sync_copy(zeros_hbm, od_sh.at[pl.ds(row0, ROWS_PER_TILE)])
    pltpu.sync_copy(zeros_hbm, id_sh.at[pl.ds(row0, ROWS_PER_TILE)])
    plsc.subcore_barrier()

    @pl.loop(0, NCHUNKS)
    def _(j):
        pltpu.async_copy(ones_v, od_sh.at[sidx.at[j]], sem_o, add=True)
        pltpu.async_copy(ones_v, id_sh.at[didx.at[j]], sem_i, add=True)

        @pl.when(j >= 4)
        def _():
            pltpu.make_async_copy(ones_v, od_sh.at[sidx.at[j - 4]], sem_o).wait()
            pltpu.make_async_copy(ones_v, id_sh.at[didx.at[j - 4]], sem_i).wait()

    @pl.loop(NCHUNKS - 4, NCHUNKS)
    def _(j):
        pltpu.make_async_copy(ones_v, od_sh.at[sidx.at[j]], sem_o).wait()
        pltpu.make_async_copy(ones_v, id_sh.at[didx.at[j]], sem_i).wait()

    plsc.subcore_barrier()
    pltpu.sync_copy(od_sh.at[pl.ds(row0, ROWS_PER_TILE)],
                    od_out.at[c, pl.ds(row0, ROWS_PER_TILE)])
    pltpu.sync_copy(id_sh.at[pl.ds(row0, ROWS_PER_TILE)],
                    id_out.at[c, pl.ds(row0, ROWS_PER_TILE)])


_NBUF = 2


def _make_sc_aggregate(width):
    @functools.partial(
        pl.kernel,
        out_type=jax.ShapeDtypeStruct((NC, N_PAD, width), jnp.float32),
        mesh=_mesh,
        scratch_types=[
            pltpu.VMEM((NCHUNKS, CHUNK), jnp.int32),
            pltpu.VMEM((NCHUNKS, CHUNK), jnp.int32),
            pltpu.VMEM((CHUNK, width), jnp.float32),
            pltpu.VMEM((CHUNK, width), jnp.float32),
            pltpu.SemaphoreType.DMA,
            pltpu.SemaphoreType.DMA,
            pltpu.SemaphoreType.DMA,
            pltpu.SemaphoreType.DMA,
            pltpu.VMEM_SHARED((N_PAD, width), jnp.float32),
        ],
        compiler_params=_sc_params,
    )
    def _sc_aggregate(h_hbm, src_hbm, dst_hbm, zeros_hbm, out_hbm,
                      sidx, didx, b0, b1,
                      g0, g1, s0, s1, agg_sh):
        bufs = [b0, b1]
        gsems = [g0, g1]
        ssems = [s0, s1]
        c = lax.axis_index("c")
        s = lax.axis_index("s")
        wid = s * NC + c

        pltpu.sync_copy(src_hbm.at[wid], sidx)
        pltpu.sync_copy(dst_hbm.at[wid], didx)
        row0 = s * ROWS_PER_TILE
        pltpu.sync_copy(zeros_hbm, agg_sh.at[pl.ds(row0, ROWS_PER_TILE)])
        plsc.subcore_barrier()

        for b in range(_NBUF):
            pltpu.async_copy(h_hbm.at[sidx.at[b]], bufs[b], gsems[b])

        @pl.loop(0, NCHUNKS - 1, step=_NBUF)
        def _(jj):
            for b in range(_NBUF):
                pltpu.make_async_copy(h_hbm.at[sidx.at[jj + b]],
                                      bufs[b], gsems[b]).wait()
                pltpu.async_copy(bufs[b], agg_sh.at[didx.at[jj + b]],
                                 ssems[b], add=True)
            for b in range(_NBUF):
                pltpu.make_async_copy(bufs[b], agg_sh.at[didx.at[jj + b]],
                                      ssems[b]).wait()

                @pl.when(jj + _NBUF + b < NCHUNKS)
                def _(b=b, jj=jj):
                    pltpu.async_copy(h_hbm.at[sidx.at[jj + _NBUF + b]],
                                     bufs[b], gsems[b])

        pltpu.make_async_copy(h_hbm.at[sidx.at[NCHUNKS - 1]],
                              bufs[0], gsems[0]).wait()
        pltpu.sync_copy(bufs[0], agg_sh.at[didx.at[NCHUNKS - 1]], add=True)

        plsc.subcore_barrier()
        pltpu.sync_copy(agg_sh.at[pl.ds(row0, ROWS_PER_TILE)],
                        out_hbm.at[c, pl.ds(row0, ROWS_PER_TILE)])

    return _sc_aggregate


_sc_aggregate_h = _make_sc_aggregate(HIDDEN)
_sc_aggregate_c = _make_sc_aggregate(CLS_PAD)


_ROWS_BLK = 1000
_GRID = N_NODES // _ROWS_BLK


def _norm_from_partials(p_ref):
    deg = p_ref[0][:, :1] + p_ref[1][:, :1]
    return lax.rsqrt(jnp.maximum(deg, 1.0))


def _tc_scale_body(feat_ref, odp_ref, h1_ref):
    h1_ref[...] = feat_ref[...] * _norm_from_partials(odp_ref)


def _tc_scale(features, odeg_p):
    return pl.pallas_call(
        _tc_scale_body,
        grid=(_GRID,),
        in_specs=[
            pl.BlockSpec((_ROWS_BLK, IN_FEATS), lambda i: (i, 0)),
            pl.BlockSpec((NC, _ROWS_BLK, 16), lambda i: (0, i, 0)),
        ],
        out_specs=pl.BlockSpec((_ROWS_BLK, IN_FEATS), lambda i: (i, 0)),
        out_shape=jax.ShapeDtypeStruct((N_NODES, IN_FEATS), jnp.float32),
    )(features, odeg_p)


def _tc_layer1_body(p1_ref, idp_ref, odp_ref, w1_ref, b1_ref, w2_ref, y_ref):
    agg = (p1_ref[0] + p1_ref[1]) * _norm_from_partials(idp_ref)
    x1 = jnp.dot(agg, w1_ref[...], preferred_element_type=jnp.float32,
                 precision=lax.Precision.HIGHEST)
    x1 = jnp.maximum(x1 + b1_ref[...], 0.0)
    x1 = x1 * _norm_from_partials(odp_ref)
    y_ref[...] = jnp.dot(x1, w2_ref[...], preferred_element_type=jnp.float32,
                         precision=lax.Precision.HIGHEST)


def _tc_layer1(p1, ideg_p, odeg_p, W1, b1, W2p):
    return pl.pallas_call(
        _tc_layer1_body,
        grid=(_GRID,),
        in_specs=[
            pl.BlockSpec((NC, _ROWS_BLK, HIDDEN), lambda i: (0, i, 0)),
            pl.BlockSpec((NC, _ROWS_BLK, 16), lambda i: (0, i, 0)),
            pl.BlockSpec((NC, _ROWS_BLK, 16), lambda i: (0, i, 0)),
            pl.BlockSpec((IN_FEATS, HIDDEN), lambda i: (0, 0)),
            pl.BlockSpec((1, HIDDEN), lambda i: (0, 0)),
            pl.BlockSpec((HIDDEN, CLS_PAD), lambda i: (0, 0)),
        ],
        out_specs=pl.BlockSpec((_ROWS_BLK, CLS_PAD), lambda i: (i, 0)),
        out_shape=jax.ShapeDtypeStruct((N_NODES, CLS_PAD), jnp.float32),
    )(p1, ideg_p, odeg_p, W1, b1, W2p)


def _tc_layer2_body(p2_ref, idp_ref, b2_ref, out_ref):
    agg = (p2_ref[0] + p2_ref[1])[:, :NUM_CLASSES]
    out_ref[...] = agg * _norm_from_partials(idp_ref) + b2_ref[...]


def _tc_layer2(p2, ideg_p, b2):
    return pl.pallas_call(
        _tc_layer2_body,
        grid=(_GRID,),
        in_specs=[
            pl.BlockSpec((NC, _ROWS_BLK, CLS_PAD), lambda i: (0, i, 0)),
            pl.BlockSpec((NC, _ROWS_BLK, 16), lambda i: (0, i, 0)),
            pl.BlockSpec((1, NUM_CLASSES), lambda i: (0, 0)),
        ],
        out_specs=pl.BlockSpec((_ROWS_BLK, NUM_CLASSES), lambda i: (i, 0)),
        out_shape=jax.ShapeDtypeStruct((N_NODES, NUM_CLASSES), jnp.float32),
    )(p2, ideg_p, b2)


def kernel(features, edge_index, W1, b1, W2, b2):
    src = edge_index[0].reshape(NW, NCHUNKS, CHUNK)
    dst = edge_index[1].reshape(NW, NCHUNKS, CHUNK)

    ones16 = jnp.ones((CHUNK, 16), jnp.float32)
    zeros16 = jnp.zeros((ROWS_PER_TILE, 16), jnp.float32)
    zeros_h = jnp.zeros((ROWS_PER_TILE, HIDDEN), jnp.float32)
    zeros_c = jnp.zeros((ROWS_PER_TILE, CLS_PAD), jnp.float32)
    W2p = jnp.pad(W2, ((0, 0), (0, CLS_PAD - NUM_CLASSES)))

    odeg_p, ideg_p = _sc_degrees(src, dst, ones16, zeros16)

    h1 = _tc_scale(features, odeg_p)
    p1 = _sc_aggregate_h(h1, src, dst, zeros_h)
    y = _tc_layer1(p1, ideg_p, odeg_p, W1, b1.reshape(1, HIDDEN), W2p)
    p2 = _sc_aggregate_c(y, src, dst, zeros_c)
    out = _tc_layer2(p2, ideg_p, b2.reshape(1, NUM_CLASSES))
    return out
